# baseline (speedup 1.0000x reference)
import dataclasses
import functools

_HI = "highest"

import jax
import jax.numpy as jnp
from jax import lax
from jax.experimental import pallas as pl
from jax.experimental.pallas import tpu as pltpu
from jax.experimental.pallas import tpu_sc as plsc

N = 10000
C = 8
E = 640000
K = 5
FFN = 200
BOT = 16

NW = 32
CHUNK = 80
EPW = E // NW
CPW = EPW // CHUNK
NBLK = NW * CPW
D = 48
DW = D // 2

NBUF = 2

BK = 16000
NKS = (N * C) // BK
BM = 8000
NMS = (N * C) // BM


def _elu(x):
    return jnp.where(x > 0, x, jnp.exp(jnp.minimum(x, 0.0)) - 1.0)


def _prologue_body(x_ref, g_ref, rootT_ref, b_ref, xg_ref, root_ref):
    x = x_ref[...]
    xg_ref[...] = jnp.dot(
        x, g_ref[...], preferred_element_type=jnp.float32, precision=_HI
    ).astype(jnp.bfloat16)
    root_ref[...] = (
        jnp.dot(x, rootT_ref[...], preferred_element_type=jnp.float32, precision=_HI)
        + b_ref[...]
    )


def _prologue(x, g_pad, rootT, bias2):
    return pl.pallas_call(
        _prologue_body,
        out_shape=(
            jax.ShapeDtypeStruct((N, D), jnp.bfloat16),
            jax.ShapeDtypeStruct((N, C), jnp.float32),
        ),
    )(x, g_pad, rootT, bias2)


_NR = (N * C) // 128


def _combine_body(parts_ref, cnt_ref, root_ref, res_ref, o_ref, *, apply_elu):
    acc = parts_ref[0]
    cnt = cnt_ref[0]
    for i in range(1, NW):
        acc = acc + parts_ref[i]
        cnt = cnt + cnt_ref[i]
    inv = 1.0 / jnp.maximum(cnt, 1.0)
    lane = lax.broadcasted_iota(jnp.int32, (16, 128), 1)
    grp = lax.broadcasted_iota(jnp.int32, (16, 128), 0)
    rep = jnp.where(lane // C == grp, 1.0, 0.0).astype(jnp.float32)
    inv128 = jnp.dot(inv, rep, preferred_element_type=jnp.float32, precision=_HI)
    pre = acc * inv128 + root_ref[...] + res_ref[...]
    o_ref[...] = _elu(pre) if apply_elu else pre


def _combine(parts, cnt, root, res, apply_elu):
    out = pl.pallas_call(
        functools.partial(_combine_body, apply_elu=apply_elu),
        out_shape=jax.ShapeDtypeStruct((_NR, 128), jnp.float32),
    )(
        parts.reshape(NW, _NR, 128),
        cnt.reshape(NW, _NR, 16),
        root.reshape(_NR, 128),
        res.reshape(_NR, 128),
    )
    return out.reshape(N, C)


def _encfc_body(w_ref, x_ref, b_ref, w2_ref, b2_ref, wd1_ref, bd1_ref, o_ref):
    k = pl.program_id(0)

    @pl.when(k == 0)
    def _():
        o_ref[...] = jnp.zeros_like(o_ref)

    o_ref[...] += jnp.dot(
        w_ref[...], x_ref[...], preferred_element_type=jnp.float32, precision=_HI
    )

    @pl.when(k == NKS - 1)
    def _():
        h1 = _elu(o_ref[...] + b_ref[...])
        z = (
            jnp.dot(w2_ref[...], h1, preferred_element_type=jnp.float32, precision=_HI)
            + b2_ref[...]
        )
        o_ref[...] = _elu(
            jnp.dot(wd1_ref[...], z, preferred_element_type=jnp.float32, precision=_HI)
            + bd1_ref[...]
        )


def _encfc(w, flat, b2d, w2, b2, wd1, bd1):
    full = lambda shape: pl.BlockSpec(shape, lambda k: (0,) * len(shape))
    return pl.pallas_call(
        _encfc_body,
        grid=(NKS,),
        in_specs=[
            pl.BlockSpec((FFN, BK), lambda k: (0, k)),
            pl.BlockSpec((BK, 1), lambda k: (k, 0)),
            full((FFN, 1)),
            full((BOT, FFN)),
            full((BOT, 1)),
            full((FFN, BOT)),
            full((FFN, 1)),
        ],
        out_specs=pl.BlockSpec((FFN, 1), lambda k: (0, 0)),
        out_shape=jax.ShapeDtypeStruct((FFN, 1), jnp.float32),
    )(w, flat, b2d, w2, b2, wd1, bd1)


def _gemv4_body(w_ref, d_ref, b_ref, o_ref):
    o_ref[...] = _elu(
        jnp.dot(w_ref[...], d_ref[...], preferred_element_type=jnp.float32, precision=_HI)
        + b_ref[...]
    )


def _gemv4(w, d1, b2d):
    return pl.pallas_call(
        _gemv4_body,
        grid=(NMS,),
        in_specs=[
            pl.BlockSpec((BM, FFN), lambda m: (m, 0)),
            pl.BlockSpec((FFN, 1), lambda m: (0, 0)),
            pl.BlockSpec((BM, 1), lambda m: (m, 0)),
        ],
        out_specs=pl.BlockSpec((BM, 1), lambda m: (m, 0)),
        out_shape=jax.ShapeDtypeStruct((N * C, 1), jnp.float32),
    )(w, d1, b2d)


def _sc_conv(table, eidx, ew, params, with_count):
    mesh = plsc.VectorSubcoreMesh(core_axis_name="c", subcore_axis_name="s")

    out_type = [jax.ShapeDtypeStruct((NW, N * C), jnp.float32)]
    if with_count:
        out_type.append(jax.ShapeDtypeStruct((NW, N), jnp.float32))

    def body(table_h, eidx_h, ew_h, params_h, *rest):
        if with_count:
            parts_h, cnt_h = rest[0], rest[1]
            scr = rest[2:]
        else:
            parts_h = rest[0]
            cnt_h = None
            scr = rest[1:]
        agg_v, cnt_v, tab_sh = scr[0], scr[1], scr[-1]
        p = 2
        s_bufs = scr[p:p + NBUF]; p += NBUF
        d_bufs = scr[p:p + NBUF]; p += NBUF
        w_bufs = scr[p:p + NBUF]; p += NBUF
        rows_bufs = scr[p:p + NBUF]; p += NBUF
        par_v = scr[p]; p += 1
        sems_e = scr[p:p + NBUF]; p += NBUF
        sems_r = scr[p:p + NBUF]

        cid = lax.axis_index("c")
        sid = lax.axis_index("s")
        wid = sid * 2 + cid

        zero16 = jnp.zeros((16,), jnp.float32)

        @pl.loop(0, N * C, step=16, unroll=8)
        def _(i):
            agg_v[pl.ds(i, 16)] = zero16

        if with_count:
            @pl.loop(0, N, step=16, unroll=8)
            def _(i):
                cnt_v[pl.ds(i, 16)] = zero16

        pltpu.sync_copy(params_h, par_v)

        base_e = wid * EPW

        def e_copies(j, b):
            base = base_e + j * CHUNK
            return (
                pltpu.make_async_copy(
                    eidx_h.at[0, pl.ds(base, CHUNK)], s_bufs[b], sems_e[b]
                ),
                pltpu.make_async_copy(
                    eidx_h.at[1, pl.ds(base, CHUNK)], d_bufs[b], sems_e[b]
                ),
                pltpu.make_async_copy(
                    ew_h.at[pl.ds(base, CHUNK)], w_bufs[b], sems_e[b]
                ),
            )

        def e_start(j, b):
            for c in e_copies(j, b):
                c.start()

        def e_wait(j, b):
            for c in e_copies(j, b):
                c.wait()

        SLICE = N // 16
        pltpu.sync_copy(
            table_h.at[pl.ds(sid * SLICE, SLICE)],
            tab_sh.at[pl.ds(sid * SLICE, SLICE)],
        )
        plsc.subcore_barrier()

        def r_copy(b):
            return pltpu.make_async_copy(
                tab_sh.at[s_bufs[b]], rows_bufs[b], sems_r[b]
            )

        for t in range(NBUF):
            e_start(t, t)
        for t in range(NBUF - 1):
            e_wait(t, t)
            r_copy(t).start()

        iota16 = lax.iota(jnp.int32, 16)
        mus = [par_v[k] for k in range(K)]
        avs = [par_v[K + k] for k in range(K)]
        ones16 = jnp.full((16,), 1.0, jnp.float32)
        pairvs = [
            [jnp.full((16,), k * (C // 2) + p, jnp.int32)
             for p in range(C // 2)]
            for k in range(K)
        ]

        def do_chunk(j, b):
            nb = (b + NBUF - 1) % NBUF

            @pl.when(j + NBUF - 1 < CPW)
            def _():
                e_wait(j + NBUF - 1, nb)
                r_copy(nb).start()

            r_copy(b).wait()

            dstvs = [d_bufs[b][pl.ds(t * 16, 16)] for t in range(5)]
            wvs = [w_bufs[b][pl.ds(t * 16, 16)] for t in range(5)]

            @pl.when(j + NBUF < CPW)
            def _():
                e_start(j + NBUF, b)

            rows = rows_bufs[b]
            for t in range(5):
                wv = wvs[t]
                dstv = dstvs[t]
                gs = []
                for k in range(K):
                    d = wv - mus[k]
                    gs.append(jnp.exp(d * d * avs[k]))
                rid = iota16 + (t * 16)
                d8 = dstv * C
                accs = [None] * C
                for k in range(K):
                    for p in range(C // 2):
                        w32 = plsc.load_gather(rows, [rid, pairvs[k][p]])
                        bf = plsc.bitcast(w32, jnp.bfloat16)
                        lo, hi = plsc.unpack(
                            bf, format=plsc.PackFormat.INTERLEAVED,
                            preferred_element_type=jnp.float32,
                        )
                        for c, v in ((2 * p, lo), (2 * p + 1, hi)):
                            term = gs[k] * v
                            accs[c] = term if accs[c] is None else accs[c] + term
                for c in range(C):
                    plsc.addupdate_scatter(agg_v, [d8 + c], accs[c])
                if with_count:
                    plsc.addupdate_scatter(cnt_v, [dstv], ones16)

        tail = CPW % NBUF
        main = CPW - tail

        @pl.loop(0, main, step=NBUF)
        def _(i):
            for t in range(NBUF):
                do_chunk(i + t, t)

        for t in range(tail):
            do_chunk(main + t, t)

        pltpu.sync_copy(agg_v, parts_h.at[wid])
        if with_count:
            pltpu.sync_copy(cnt_v, cnt_h.at[wid])

    scratch = (
        [
            pltpu.VMEM((N * C,), jnp.float32),
            pltpu.VMEM((N,), jnp.float32),
        ]
        + [pltpu.VMEM((CHUNK,), jnp.int32) for _ in range(NBUF)]
        + [pltpu.VMEM((CHUNK,), jnp.int32) for _ in range(NBUF)]
        + [pltpu.VMEM((CHUNK,), jnp.float32) for _ in range(NBUF)]
        + [pltpu.VMEM((CHUNK, DW), jnp.int32) for _ in range(NBUF)]
        + [pltpu.VMEM((2 * K, 16), jnp.float32)]
        + [pltpu.SemaphoreType.DMA for _ in range(2 * NBUF)]
        + [pltpu.VMEM_SHARED((N, DW), jnp.int32)]
    )

    cp = pltpu.CompilerParams()
    if "needs_layout_passes" in pltpu.CompilerParams.__dataclass_fields__:
        cp = dataclasses.replace(cp, needs_layout_passes=False)
    if "use_tc_tiling_on_sc" in pltpu.CompilerParams.__dataclass_fields__:
        cp = dataclasses.replace(cp, use_tc_tiling_on_sc=False)

    run = pl.kernel(
        body, out_type=tuple(out_type), mesh=mesh, scratch_types=scratch,
        compiler_params=cp,
    )
    return run(table, eidx, ew, params)


def _pack_params(mu, sigma):
    a = -0.5 / (1e-15 + sigma[:, 0] ** 2)
    m = mu[:, 0]
    both = jnp.concatenate([m, a], 0)
    return jnp.tile(both[:, None], (1, 16)).astype(jnp.float32)


def kernel(x, edge_index, edge_weight,
           enc_g, enc_mu, enc_sigma, enc_root, enc_bias,
           W_enc1, b_enc1, W_enc2, b_enc2,
           W_dec1, b_dec1, W_dec2, b_dec2,
           dec_g, dec_mu, dec_sigma, dec_root, dec_bias):
    eidx = edge_index.astype(jnp.int32)
    ew = edge_weight[:, 0].astype(jnp.float32)

    pad = ((0, 0), (0, D - K * C))
    params1 = _pack_params(enc_mu, enc_sigma)
    params2 = _pack_params(dec_mu, dec_sigma)

    xg1, root1 = _prologue(
        x, jnp.pad(enc_g, pad), enc_root.T, enc_bias[None, :]
    )
    xg1b = lax.bitcast_convert_type(xg1.reshape(N, DW, 2), jnp.int32)
    parts1, cnt = _sc_conv(xg1b, eidx, ew, params1, with_count=True)
    xe = _combine(parts1, cnt, root1, x, apply_elu=True)

    flat = xe.reshape(N * C, 1)
    d1 = _encfc(W_enc1, flat, b_enc1[:, None], W_enc2, b_enc2[:, None],
                W_dec1, b_dec1[:, None])
    xd = _gemv4(W_dec2, d1, b_dec2[:, None]).reshape(N, C)

    xg2, root2 = _prologue(
        xd, jnp.pad(dec_g, pad), dec_root.T, dec_bias[None, :]
    )
    xg2b = lax.bitcast_convert_type(xg2.reshape(N, DW, 2), jnp.int32)
    (parts2,) = _sc_conv(xg2b, eidx, ew, params2, with_count=False)
    out = _combine(parts2, cnt, root2, xd, apply_elu=False)
    return out


# speedup vs baseline: 1.1137x; 1.1137x over previous
import dataclasses
import functools

_HI = "highest"

import jax
import jax.numpy as jnp
from jax import lax
from jax.experimental import pallas as pl
from jax.experimental.pallas import tpu as pltpu
from jax.experimental.pallas import tpu_sc as plsc

N = 10000
C = 8
E = 640000
K = 5
FFN = 200
BOT = 16

NW = 32
CHUNK = 80
EPW = E // NW
CPW = EPW // CHUNK
NBLK = NW * CPW
D = 48
DW = D // 2

NBUF = 2

BK = 16000
NKS = (N * C) // BK
BM = 8000
NMS = (N * C) // BM


def _elu(x):
    return jnp.where(x > 0, x, jnp.exp(jnp.minimum(x, 0.0)) - 1.0)


def _prologue_body(x_ref, g_ref, rootT_ref, b_ref, xg_ref, root_ref):
    x = x_ref[...]
    xg_ref[...] = jnp.dot(
        x, g_ref[...], preferred_element_type=jnp.float32
    ).astype(jnp.bfloat16)
    root_ref[...] = (
        jnp.dot(x, rootT_ref[...], preferred_element_type=jnp.float32)
        + b_ref[...]
    )


def _prologue(x, g_pad, rootT, bias2):
    return pl.pallas_call(
        _prologue_body,
        out_shape=(
            jax.ShapeDtypeStruct((N, D), jnp.bfloat16),
            jax.ShapeDtypeStruct((N, C), jnp.float32),
        ),
    )(x, g_pad, rootT, bias2)


_NR = (N * C) // 128


def _combine_body(parts_ref, cnt_ref, root_ref, res_ref, o_ref, *, apply_elu):
    acc = parts_ref[0]
    cnt = cnt_ref[0]
    for i in range(1, NW):
        acc = acc + parts_ref[i]
        cnt = cnt + cnt_ref[i]
    inv = 1.0 / jnp.maximum(cnt, 1.0)
    lane = lax.broadcasted_iota(jnp.int32, (16, 128), 1)
    grp = lax.broadcasted_iota(jnp.int32, (16, 128), 0)
    rep = jnp.where(lane // C == grp, 1.0, 0.0).astype(jnp.float32)
    inv128 = jnp.dot(inv, rep, preferred_element_type=jnp.float32)
    pre = acc * inv128 + root_ref[...] + res_ref[...]
    o_ref[...] = _elu(pre) if apply_elu else pre


def _combine(parts, cnt, root, res, apply_elu):
    out = pl.pallas_call(
        functools.partial(_combine_body, apply_elu=apply_elu),
        out_shape=jax.ShapeDtypeStruct((_NR, 128), jnp.float32),
    )(
        parts.reshape(NW, _NR, 128),
        cnt.reshape(NW, _NR, 16),
        root.reshape(_NR, 128),
        res.reshape(_NR, 128),
    )
    return out.reshape(N, C)


def _encfc_body(w_ref, x_ref, b_ref, w2_ref, b2_ref, wd1_ref, bd1_ref, o_ref):
    k = pl.program_id(0)

    @pl.when(k == 0)
    def _():
        o_ref[...] = jnp.zeros_like(o_ref)

    o_ref[...] += jnp.dot(
        w_ref[...], x_ref[...], preferred_element_type=jnp.float32
    )

    @pl.when(k == NKS - 1)
    def _():
        h1 = _elu(o_ref[...] + b_ref[...])
        z = (
            jnp.dot(w2_ref[...], h1, preferred_element_type=jnp.float32)
            + b2_ref[...]
        )
        o_ref[...] = _elu(
            jnp.dot(wd1_ref[...], z, preferred_element_type=jnp.float32)
            + bd1_ref[...]
        )


def _encfc(w, flat, b2d, w2, b2, wd1, bd1):
    full = lambda shape: pl.BlockSpec(shape, lambda k: (0,) * len(shape))
    return pl.pallas_call(
        _encfc_body,
        grid=(NKS,),
        in_specs=[
            pl.BlockSpec((FFN, BK), lambda k: (0, k)),
            pl.BlockSpec((BK, 1), lambda k: (k, 0)),
            full((FFN, 1)),
            full((BOT, FFN)),
            full((BOT, 1)),
            full((FFN, BOT)),
            full((FFN, 1)),
        ],
        out_specs=pl.BlockSpec((FFN, 1), lambda k: (0, 0)),
        out_shape=jax.ShapeDtypeStruct((FFN, 1), jnp.float32),
    )(w, flat, b2d, w2, b2, wd1, bd1)


def _gemv4_body(w_ref, d_ref, b_ref, o_ref):
    o_ref[...] = _elu(
        jnp.dot(w_ref[...], d_ref[...], preferred_element_type=jnp.float32)
        + b_ref[...]
    )


def _gemv4(w, d1, b2d):
    return pl.pallas_call(
        _gemv4_body,
        grid=(NMS,),
        in_specs=[
            pl.BlockSpec((BM, FFN), lambda m: (m, 0)),
            pl.BlockSpec((FFN, 1), lambda m: (0, 0)),
            pl.BlockSpec((BM, 1), lambda m: (m, 0)),
        ],
        out_specs=pl.BlockSpec((BM, 1), lambda m: (m, 0)),
        out_shape=jax.ShapeDtypeStruct((N * C, 1), jnp.float32),
    )(w, d1, b2d)


def _sc_conv(table, eidx, ew, params, with_count):
    mesh = plsc.VectorSubcoreMesh(core_axis_name="c", subcore_axis_name="s")

    out_type = [jax.ShapeDtypeStruct((NW, N * C), jnp.float32)]
    if with_count:
        out_type.append(jax.ShapeDtypeStruct((NW, N), jnp.float32))

    def body(table_h, eidx_h, ew_h, params_h, *rest):
        if with_count:
            parts_h, cnt_h = rest[0], rest[1]
            scr = rest[2:]
        else:
            parts_h = rest[0]
            cnt_h = None
            scr = rest[1:]
        agg_v, cnt_v, tab_sh = scr[0], scr[1], scr[-1]
        p = 2
        s_bufs = scr[p:p + NBUF]; p += NBUF
        d_bufs = scr[p:p + NBUF]; p += NBUF
        w_bufs = scr[p:p + NBUF]; p += NBUF
        rows_bufs = scr[p:p + NBUF]; p += NBUF
        par_v = scr[p]; p += 1
        sems_e = scr[p:p + NBUF]; p += NBUF
        sems_r = scr[p:p + NBUF]

        cid = lax.axis_index("c")
        sid = lax.axis_index("s")
        wid = sid * 2 + cid

        zero16 = jnp.zeros((16,), jnp.float32)

        @pl.loop(0, N * C, step=16, unroll=8)
        def _(i):
            agg_v[pl.ds(i, 16)] = zero16

        if with_count:
            @pl.loop(0, N, step=16, unroll=8)
            def _(i):
                cnt_v[pl.ds(i, 16)] = zero16

        pltpu.sync_copy(params_h, par_v)

        base_e = wid * EPW

        def e_copies(j, b):
            base = base_e + j * CHUNK
            return (
                pltpu.make_async_copy(
                    eidx_h.at[0, pl.ds(base, CHUNK)], s_bufs[b], sems_e[b]
                ),
                pltpu.make_async_copy(
                    eidx_h.at[1, pl.ds(base, CHUNK)], d_bufs[b], sems_e[b]
                ),
                pltpu.make_async_copy(
                    ew_h.at[pl.ds(base, CHUNK)], w_bufs[b], sems_e[b]
                ),
            )

        def e_start(j, b):
            for c in e_copies(j, b):
                c.start()

        def e_wait(j, b):
            for c in e_copies(j, b):
                c.wait()

        SLICE = N // 16
        pltpu.sync_copy(
            table_h.at[pl.ds(sid * SLICE, SLICE)],
            tab_sh.at[pl.ds(sid * SLICE, SLICE)],
        )
        plsc.subcore_barrier()

        def r_copy(b):
            return pltpu.make_async_copy(
                tab_sh.at[s_bufs[b]], rows_bufs[b], sems_r[b]
            )

        for t in range(NBUF):
            e_start(t, t)
        for t in range(NBUF - 1):
            e_wait(t, t)
            r_copy(t).start()

        iota16 = lax.iota(jnp.int32, 16)
        mus = [par_v[k] for k in range(K)]
        avs = [par_v[K + k] for k in range(K)]
        ones16 = jnp.full((16,), 1.0, jnp.float32)
        pairvs = [
            [jnp.full((16,), k * (C // 2) + p, jnp.int32)
             for p in range(C // 2)]
            for k in range(K)
        ]

        def do_chunk(j, b):
            nb = (b + NBUF - 1) % NBUF

            @pl.when(j + NBUF - 1 < CPW)
            def _():
                e_wait(j + NBUF - 1, nb)
                r_copy(nb).start()

            r_copy(b).wait()

            dstvs = [d_bufs[b][pl.ds(t * 16, 16)] for t in range(5)]
            wvs = [w_bufs[b][pl.ds(t * 16, 16)] for t in range(5)]

            @pl.when(j + NBUF < CPW)
            def _():
                e_start(j + NBUF, b)

            rows = rows_bufs[b]
            for t in range(5):
                wv = wvs[t]
                dstv = dstvs[t]
                gs = []
                for k in range(K):
                    d = wv - mus[k]
                    gs.append(jnp.exp(d * d * avs[k]))
                rid = iota16 + (t * 16)
                d8 = dstv * C
                accs = [None] * C
                for k in range(K):
                    for p in range(C // 2):
                        w32 = plsc.load_gather(rows, [rid, pairvs[k][p]])
                        bf = plsc.bitcast(w32, jnp.bfloat16)
                        lo, hi = plsc.unpack(
                            bf, format=plsc.PackFormat.INTERLEAVED,
                            preferred_element_type=jnp.float32,
                        )
                        for c, v in ((2 * p, lo), (2 * p + 1, hi)):
                            term = gs[k] * v
                            accs[c] = term if accs[c] is None else accs[c] + term
                for c in range(C):
                    plsc.addupdate_scatter(agg_v, [d8 + c], accs[c])
                if with_count:
                    plsc.addupdate_scatter(cnt_v, [dstv], ones16)

        tail = CPW % NBUF
        main = CPW - tail

        @pl.loop(0, main, step=NBUF)
        def _(i):
            for t in range(NBUF):
                do_chunk(i + t, t)

        for t in range(tail):
            do_chunk(main + t, t)

        pltpu.sync_copy(agg_v, parts_h.at[wid])
        if with_count:
            pltpu.sync_copy(cnt_v, cnt_h.at[wid])

    scratch = (
        [
            pltpu.VMEM((N * C,), jnp.float32),
            pltpu.VMEM((N,), jnp.float32),
        ]
        + [pltpu.VMEM((CHUNK,), jnp.int32) for _ in range(NBUF)]
        + [pltpu.VMEM((CHUNK,), jnp.int32) for _ in range(NBUF)]
        + [pltpu.VMEM((CHUNK,), jnp.float32) for _ in range(NBUF)]
        + [pltpu.VMEM((CHUNK, DW), jnp.int32) for _ in range(NBUF)]
        + [pltpu.VMEM((2 * K, 16), jnp.float32)]
        + [pltpu.SemaphoreType.DMA for _ in range(2 * NBUF)]
        + [pltpu.VMEM_SHARED((N, DW), jnp.int32)]
    )

    cp = pltpu.CompilerParams()
    if "needs_layout_passes" in pltpu.CompilerParams.__dataclass_fields__:
        cp = dataclasses.replace(cp, needs_layout_passes=False)
    if "use_tc_tiling_on_sc" in pltpu.CompilerParams.__dataclass_fields__:
        cp = dataclasses.replace(cp, use_tc_tiling_on_sc=False)

    run = pl.kernel(
        body, out_type=tuple(out_type), mesh=mesh, scratch_types=scratch,
        compiler_params=cp,
    )
    return run(table, eidx, ew, params)


def _pack_params(mu, sigma):
    a = -0.5 / (1e-15 + sigma[:, 0] ** 2)
    m = mu[:, 0]
    both = jnp.concatenate([m, a], 0)
    return jnp.tile(both[:, None], (1, 16)).astype(jnp.float32)


def kernel(x, edge_index, edge_weight,
           enc_g, enc_mu, enc_sigma, enc_root, enc_bias,
           W_enc1, b_enc1, W_enc2, b_enc2,
           W_dec1, b_dec1, W_dec2, b_dec2,
           dec_g, dec_mu, dec_sigma, dec_root, dec_bias):
    eidx = edge_index.astype(jnp.int32)
    ew = edge_weight[:, 0].astype(jnp.float32)

    pad = ((0, 0), (0, D - K * C))
    params1 = _pack_params(enc_mu, enc_sigma)
    params2 = _pack_params(dec_mu, dec_sigma)

    xg1, root1 = _prologue(
        x, jnp.pad(enc_g, pad), enc_root.T, enc_bias[None, :]
    )
    xg1b = lax.bitcast_convert_type(xg1.reshape(N, DW, 2), jnp.int32)
    parts1, cnt = _sc_conv(xg1b, eidx, ew, params1, with_count=True)
    xe = _combine(parts1, cnt, root1, x, apply_elu=True)

    flat = xe.reshape(N * C, 1)
    d1 = _encfc(W_enc1, flat, b_enc1[:, None], W_enc2, b_enc2[:, None],
                W_dec1, b_dec1[:, None])
    xd = _gemv4(W_dec2, d1, b_dec2[:, None]).reshape(N, C)

    xg2, root2 = _prologue(
        xd, jnp.pad(dec_g, pad), dec_root.T, dec_bias[None, :]
    )
    xg2b = lax.bitcast_convert_type(xg2.reshape(N, DW, 2), jnp.int32)
    (parts2,) = _sc_conv(xg2b, eidx, ew, params2, with_count=False)
    out = _combine(parts2, cnt, root2, xd, apply_elu=False)
    return out
